# baseline (speedup 1.0000x reference)
import functools

import jax
import jax.numpy as jnp
from jax import lax
from jax.experimental import pallas as pl
from jax.experimental.pallas import tpu as pltpu
from jax.experimental.pallas import tpu_sc as plsc

B = 4096
F_CAT = 26
F_NUM = 13
V = 100000
D = 32
F_TOT = F_CAT + F_NUM
H0 = 50048
H1 = V - H0

NVEC = B // 16


def _sc_embed(tab_t, cat_t, ct_t, wb_flat):
    info = plsc.get_sparse_core_info()
    NC, NS = info.num_cores, info.num_subcores
    NW = NC * NS
    cat_pw = (F_CAT * D) // NW
    num_pw = (F_NUM * D) // NW
    mesh = plsc.VectorSubcoreMesh(core_axis_name="c", subcore_axis_name="s")

    @functools.partial(
        pl.kernel,
        mesh=mesh,
        compiler_params=pltpu.CompilerParams(
            use_tc_tiling_on_sc=True, needs_layout_passes=False),
        out_type=jax.ShapeDtypeStruct((F_TOT, D, B), jnp.float32),
        scratch_types=[
            pltpu.VMEM((H0,), jnp.float32),
            pltpu.VMEM((H1,), jnp.float32),
            pltpu.VMEM((B,), jnp.int32),
            pltpu.VMEM((B,), jnp.float32),
            pltpu.VMEM((2 * F_NUM * D,), jnp.float32),
            pltpu.SemaphoreType.DMA,
            pltpu.SemaphoreType.DMA,
        ],
    )
    def k(tab_hbm, cat_hbm, ct_hbm, wb_hbm, out_hbm,
          h0_v, h1_v, idx_v, out_v, wb_v, sem0, sem1):
        wid = lax.axis_index("s") * NC + lax.axis_index("c")
        pltpu.sync_copy(wb_hbm, wb_v)

        def plane_fd(p):
            g = wid * cat_pw + p
            return g // D, g % D

        def fire(p, half):
            f, d = plane_fd(p)
            if half == 0:
                return pltpu.async_copy(
                    tab_hbm.at[f, d, pl.ds(0, H0)], h0_v, sem0)
            return pltpu.async_copy(
                tab_hbm.at[f, d, pl.ds(H0, H1)], h1_v, sem1)

        cp0 = fire(0, 0)
        cp1 = fire(0, 1)
        for p in range(cat_pw):
            f, d = plane_fd(p)
            pltpu.sync_copy(cat_hbm.at[f], idx_v)
            cp0.wait()

            def pass0(i, carry):
                vec = idx_v[pl.ds(i * 16, 16)]
                m = vec < H0
                g0 = plsc.load_gather(h0_v, [vec], mask=m)
                out_v[pl.ds(i * 16, 16)] = g0
                return carry

            lax.fori_loop(0, NVEC, pass0, 0)
            if p + 1 < cat_pw:
                cp0 = fire(p + 1, 0)
            cp1.wait()

            def pass1(i, carry):
                vec = idx_v[pl.ds(i * 16, 16)]
                m = vec < H0
                g1 = plsc.load_gather(h1_v, [vec - H0], mask=jnp.logical_not(m))
                prev = out_v[pl.ds(i * 16, 16)]
                out_v[pl.ds(i * 16, 16)] = jnp.where(m, prev, g1)
                return carry

            lax.fori_loop(0, NVEC, pass1, 0)
            if p + 1 < cat_pw:
                cp1 = fire(p + 1, 1)
            pltpu.sync_copy(out_v, out_hbm.at[f, d])

        ct_v = out_v
        for q in range(num_pw):
            h = wid * num_pw + q
            j = h // D
            d = h % D
            pltpu.sync_copy(ct_hbm.at[j], ct_v)
            wsp = plsc.load_gather(wb_v, [jnp.full((16,), j * D + d, jnp.int32)])
            bsp = plsc.load_gather(
                wb_v, [jnp.full((16,), F_NUM * D + j * D + d, jnp.int32)])

            def num_body(i, carry, wsp=wsp, bsp=bsp):
                cvec = ct_v[pl.ds(i * 16, 16)]
                ct_v[pl.ds(i * 16, 16)] = jnp.maximum(cvec * wsp + bsp, 0.0)
                return carry

            lax.fori_loop(0, NVEC, num_body, 0)
            pltpu.sync_copy(ct_v, out_hbm.at[F_CAT + j, d])

    return k(tab_t, cat_t, ct_t, wb_flat)


def kernel(continuous, categorical, tables, W_num, b_num):
    tab_t = tables.transpose(0, 2, 1)
    cat_t = categorical.T
    ct_t = continuous.T
    wb_flat = jnp.concatenate([W_num.reshape(-1), b_num.reshape(-1)])
    out = _sc_embed(tab_t, cat_t, ct_t, wb_flat)
    return out.transpose(2, 0, 1)
